# baseline (speedup 1.0000x reference)
import jax
import jax.numpy as jnp
from jax.experimental import pallas as pl
from jax.experimental.pallas import tpu as pltpu

_G, _K, _D, _B = 4, 8192, 32, 2048
_CC = 0.25
_BB = 64
_NB = _B // _BB


def _vq_kernel(x_ref, w_ref, w2_ref, oh_ref, emb_ref, qst_ref, loss_ref,
               perp_ref, counts_ref):
    b = pl.program_id(1)
    first = jnp.logical_and(pl.program_id(0) == 0, b == 0)

    x = x_ref[0]
    w = w_ref[0]

    x2 = jax.lax.dot_general(x * x, jnp.ones((_D, 1), jnp.float32),
                             (((1,), (0,)), ((), ())),
                             preferred_element_type=jnp.float32)

    x2x = x + x
    _C = 2048
    _NC = _K // _C
    lane = jax.lax.broadcasted_iota(jnp.int32, (1, _C), 1)

    minv = jnp.full((_BB, _C), jnp.inf, jnp.float32)
    mini = jnp.full((_BB, _C), _K, jnp.int32)
    for i in range(_NC):
        wc = w_ref[0, i * _C:(i + 1) * _C, :]
        mm2c = jax.lax.dot_general(x2x, wc, (((1,), (1,)), ((), ())),
                                   preferred_element_type=jnp.float32)
        w2c = w2_ref[0, :, i * _C:(i + 1) * _C]
        dc = (x2 + w2c) - mm2c
        ioc = lane + i * _C
        lt = dc < minv
        minv = jnp.where(lt, dc, minv)
        mini = jnp.where(lt, jnp.broadcast_to(ioc, (_BB, _C)), mini)

    rowmin = jnp.min(minv, axis=1, keepdims=True)
    idx = jnp.min(jnp.where(minv == rowmin, mini, jnp.int32(_K)), axis=1)

    iota = jax.lax.broadcasted_iota(jnp.int32, (1, _K), 1)
    oh = (iota == idx[:, None]).astype(jnp.float32)
    oh_ref[0] = oh

    q = jax.lax.dot_general(oh, w, (((1,), (0,)), ((), ())),
                            preferred_element_type=jnp.float32)
    emb_ref[0] = q
    qst_ref[0] = x + (q - x)

    @pl.when(first)
    def _():
        loss_ref[...] = jnp.zeros((1, 1), jnp.float32)
        perp_ref[...] = jnp.zeros((1, 1), jnp.float32)

    d = q - x
    loss_ref[...] += jnp.sum(d * d).reshape(1, 1)

    colsum = jax.lax.dot_general(jnp.ones((1, _BB), jnp.float32), oh,
                                 (((1,), (0,)), ((), ())),
                                 preferred_element_type=jnp.float32)

    @pl.when(b == 0)
    def _():
        counts_ref[...] = colsum

    @pl.when(b != 0)
    def _():
        counts_ref[...] += colsum

    @pl.when(b == _NB - 1)
    def _():
        p = counts_ref[...] * (1.0 / _B)
        ent = jnp.sum(p * jnp.log(p + 1e-10))
        perp_ref[...] += jnp.exp(-ent).reshape(1, 1)


def kernel(inputs, W):
    xt = jnp.transpose(inputs, (1, 0, 2))
    w2 = jnp.sum(W * W, axis=2)[:, None, :]

    grid = (_G, _NB)
    oh, emb, qst, loss_sum, perp_sum = pl.pallas_call(
        _vq_kernel,
        grid=grid,
        in_specs=[
            pl.BlockSpec((1, _BB, _D), lambda g, b: (g, b, 0)),
            pl.BlockSpec((1, _K, _D), lambda g, b: (g, 0, 0)),
            pl.BlockSpec((1, 1, _K), lambda g, b: (g, 0, 0)),
        ],
        out_specs=[
            pl.BlockSpec((1, _BB, _K), lambda g, b: (g, b, 0)),
            pl.BlockSpec((1, _BB, _D), lambda g, b: (g, b, 0)),
            pl.BlockSpec((1, _BB, _D), lambda g, b: (g, b, 0)),
            pl.BlockSpec((1, 1), lambda g, b: (0, 0)),
            pl.BlockSpec((1, 1), lambda g, b: (0, 0)),
        ],
        out_shape=[
            jax.ShapeDtypeStruct((_G, _B, _K), jnp.float32),
            jax.ShapeDtypeStruct((_G, _B, _D), jnp.float32),
            jax.ShapeDtypeStruct((_G, _B, _D), jnp.float32),
            jax.ShapeDtypeStruct((1, 1), jnp.float32),
            jax.ShapeDtypeStruct((1, 1), jnp.float32),
        ],
        scratch_shapes=[pltpu.VMEM((1, _K), jnp.float32)],
        compiler_params=pltpu.CompilerParams(
            dimension_semantics=("arbitrary", "arbitrary")),
    )(xt, W, w2)

    avg_loss = (loss_sum[0, 0] * ((1.0 + _CC) / (_B * _D))) / _G
    avg_perplexity = perp_sum[0, 0] / _G
    quantized_all = jnp.transpose(qst, (1, 0, 2))
    return (avg_loss, quantized_all, avg_perplexity, emb, oh)


# speedup vs baseline: 1.1080x; 1.1080x over previous
import jax
import jax.numpy as jnp
from jax.experimental import pallas as pl
from jax.experimental.pallas import tpu as pltpu

_G, _K, _D, _B = 4, 8192, 32, 2048
_CC = 0.25
_BB = 128
_NB = _B // _BB


def _vq_kernel(x_ref, w_ref, w2_ref, oh_ref, emb_ref, qst_ref, loss_ref,
               idx_ref):
    b = pl.program_id(1)
    first = jnp.logical_and(pl.program_id(0) == 0, b == 0)

    x = x_ref[0]
    w = w_ref[0]

    x2 = jax.lax.dot_general(x * x, jnp.ones((_D, 1), jnp.float32),
                             (((1,), (0,)), ((), ())),
                             preferred_element_type=jnp.float32)

    x2x = x + x
    _C = 2048
    _NC = _K // _C
    lane = jax.lax.broadcasted_iota(jnp.int32, (1, _C), 1)

    minv = jnp.full((_BB, _C), jnp.inf, jnp.float32)
    mini = jnp.full((_BB, _C), _K, jnp.int32)
    for i in range(_NC):
        wc = w_ref[0, i * _C:(i + 1) * _C, :]
        mm2c = jax.lax.dot_general(x2x, wc, (((1,), (1,)), ((), ())),
                                   preferred_element_type=jnp.float32)
        w2c = w2_ref[0, :, i * _C:(i + 1) * _C]
        dc = (x2 + w2c) - mm2c
        ioc = lane + i * _C
        lt = dc < minv
        minv = jnp.where(lt, dc, minv)
        mini = jnp.where(lt, jnp.broadcast_to(ioc, (_BB, _C)), mini)

    rowmin = jnp.min(minv, axis=1, keepdims=True)
    idx = jnp.min(jnp.where(minv == rowmin, mini, jnp.int32(_K)), axis=1)
    idx_ref[0, 0, :] = idx

    iota = jax.lax.broadcasted_iota(jnp.int32, (1, _K), 1)
    oh = (iota == idx[:, None]).astype(jnp.float32)
    oh_ref[0] = oh

    q = jax.lax.dot_general(oh, w, (((1,), (0,)), ((), ())),
                            preferred_element_type=jnp.float32)
    emb_ref[0] = q
    qst_ref[0] = x + (q - x)

    @pl.when(first)
    def _():
        loss_ref[...] = jnp.zeros((1, 1), jnp.float32)

    d = q - x
    loss_ref[...] += jnp.sum(d * d).reshape(1, 1)


def kernel(inputs, W):
    xt = jnp.transpose(inputs, (1, 0, 2))
    w2 = jnp.sum(W * W, axis=2)[:, None, :]

    grid = (_G, _NB)
    oh, emb, qst, loss_sum, idx = pl.pallas_call(
        _vq_kernel,
        grid=grid,
        in_specs=[
            pl.BlockSpec((1, _BB, _D), lambda g, b: (g, b, 0)),
            pl.BlockSpec((1, _K, _D), lambda g, b: (g, 0, 0)),
            pl.BlockSpec((1, 1, _K), lambda g, b: (g, 0, 0)),
        ],
        out_specs=[
            pl.BlockSpec((1, _BB, _K), lambda g, b: (g, b, 0)),
            pl.BlockSpec((1, _BB, _D), lambda g, b: (g, b, 0)),
            pl.BlockSpec((1, _BB, _D), lambda g, b: (g, b, 0)),
            pl.BlockSpec((1, 1), lambda g, b: (0, 0)),
            pl.BlockSpec((1, 1, _BB), lambda g, b: (g * _NB + b, 0, 0)),
        ],
        out_shape=[
            jax.ShapeDtypeStruct((_G, _B, _K), jnp.float32),
            jax.ShapeDtypeStruct((_G, _B, _D), jnp.float32),
            jax.ShapeDtypeStruct((_G, _B, _D), jnp.float32),
            jax.ShapeDtypeStruct((1, 1), jnp.float32),
            jax.ShapeDtypeStruct((_G * _NB, 1, _BB), jnp.int32),
        ],
        compiler_params=pltpu.CompilerParams(
            dimension_semantics=("arbitrary", "arbitrary")),
    )(xt, W, w2)

    avg_loss = (loss_sum[0, 0] * ((1.0 + _CC) / (_B * _D))) / _G
    flat = (idx.reshape(_G, _B)
            + jnp.arange(_G, dtype=jnp.int32)[:, None] * _K).reshape(-1)
    counts = jnp.zeros((_G * _K,), jnp.float32).at[flat].add(1.0)
    p = counts.reshape(_G, _K) * (1.0 / _B)
    perp = jnp.exp(-jnp.sum(p * jnp.log(p + 1e-10), axis=1))
    avg_perplexity = jnp.sum(perp) / _G
    quantized_all = jnp.transpose(qst, (1, 0, 2))
    return (avg_loss, quantized_all, avg_perplexity, emb, oh)


# speedup vs baseline: 1.3730x; 1.2392x over previous
import jax
import jax.numpy as jnp
from jax.experimental import pallas as pl
from jax.experimental.pallas import tpu as pltpu

_G, _K, _D, _B = 4, 8192, 32, 2048
_CC = 0.25
_BB = 128
_NB = _B // _BB


def _vq_kernel(x_ref, w_ref, w2_ref, oh_ref, emb_ref, qst_ref, loss_ref,
               perp_ref, counts_ref):
    b = pl.program_id(1)
    first = jnp.logical_and(pl.program_id(0) == 0, b == 0)

    x = x_ref[0]
    w = w_ref[0]

    x2 = jax.lax.dot_general(x * x, jnp.ones((_D, 1), jnp.float32),
                             (((1,), (0,)), ((), ())),
                             preferred_element_type=jnp.float32)

    x2x = x + x
    _C = 2048
    _NC = _K // _C
    lane = jax.lax.broadcasted_iota(jnp.int32, (1, _C), 1)

    minv = jnp.full((_BB, _C), jnp.inf, jnp.float32)
    mini = jnp.full((_BB, _C), _K, jnp.int32)
    for i in range(_NC):
        wc = w_ref[0, i * _C:(i + 1) * _C, :]
        mm2c = jax.lax.dot_general(x2x, wc, (((1,), (1,)), ((), ())),
                                   preferred_element_type=jnp.float32)
        w2c = w2_ref[0, :, i * _C:(i + 1) * _C]
        dc = (x2 + w2c) - mm2c
        ioc = lane + i * _C
        lt = dc < minv
        minv = jnp.where(lt, dc, minv)
        mini = jnp.where(lt, jnp.broadcast_to(ioc, (_BB, _C)), mini)

    rowmin = jnp.min(minv, axis=1, keepdims=True)
    idx = jnp.min(jnp.where(minv == rowmin, mini, jnp.int32(_K)), axis=1)

    iota = jax.lax.broadcasted_iota(jnp.int32, (1, _K), 1)
    oh = (iota == idx[:, None]).astype(jnp.float32)
    oh_ref[0] = oh

    q = jax.lax.dot_general(oh, w, (((1,), (0,)), ((), ())),
                            preferred_element_type=jnp.float32)
    emb_ref[0] = q
    qst_ref[0] = x + (q - x)

    @pl.when(first)
    def _():
        loss_ref[...] = jnp.zeros((1, 1), jnp.float32)
        perp_ref[...] = jnp.zeros((1, 1), jnp.float32)

    d = q - x
    loss_ref[...] += jnp.sum(d * d).reshape(1, 1)

    colsum = jax.lax.dot_general(jnp.ones((1, _BB), jnp.float32), oh,
                                 (((1,), (0,)), ((), ())),
                                 preferred_element_type=jnp.float32)

    @pl.when(b == 0)
    def _():
        counts_ref[...] = colsum

    @pl.when(b != 0)
    def _():
        counts_ref[...] += colsum

    @pl.when(b == _NB - 1)
    def _():
        p = counts_ref[...] * (1.0 / _B)
        ent = jnp.sum(p * jnp.log(p + 1e-10))
        perp_ref[...] += jnp.exp(-ent).reshape(1, 1)


def kernel(inputs, W):
    xt = jnp.transpose(inputs, (1, 0, 2))
    w2 = jnp.sum(W * W, axis=2)[:, None, :]

    grid = (_G, _NB)
    oh, emb, qst, loss_sum, perp_sum = pl.pallas_call(
        _vq_kernel,
        grid=grid,
        in_specs=[
            pl.BlockSpec((1, _BB, _D), lambda g, b: (g, b, 0)),
            pl.BlockSpec((1, _K, _D), lambda g, b: (g, 0, 0)),
            pl.BlockSpec((1, 1, _K), lambda g, b: (g, 0, 0)),
        ],
        out_specs=[
            pl.BlockSpec((1, _BB, _K), lambda g, b: (g, b, 0)),
            pl.BlockSpec((1, _BB, _D), lambda g, b: (g, b, 0)),
            pl.BlockSpec((1, _BB, _D), lambda g, b: (g, b, 0)),
            pl.BlockSpec((1, 1), lambda g, b: (0, 0)),
            pl.BlockSpec((1, 1), lambda g, b: (0, 0)),
        ],
        out_shape=[
            jax.ShapeDtypeStruct((_G, _B, _K), jnp.float32),
            jax.ShapeDtypeStruct((_G, _B, _D), jnp.float32),
            jax.ShapeDtypeStruct((_G, _B, _D), jnp.float32),
            jax.ShapeDtypeStruct((1, 1), jnp.float32),
            jax.ShapeDtypeStruct((1, 1), jnp.float32),
        ],
        scratch_shapes=[pltpu.VMEM((1, _K), jnp.float32)],
        compiler_params=pltpu.CompilerParams(
            dimension_semantics=("arbitrary", "arbitrary")),
    )(xt, W, w2)

    avg_loss = (loss_sum[0, 0] * ((1.0 + _CC) / (_B * _D))) / _G
    avg_perplexity = perp_sum[0, 0] / _G
    quantized_all = jnp.transpose(qst, (1, 0, 2))
    return (avg_loss, quantized_all, avg_perplexity, emb, oh)


# speedup vs baseline: 1.3842x; 1.0081x over previous
import jax
import jax.numpy as jnp
from jax.experimental import pallas as pl
from jax.experimental.pallas import tpu as pltpu

_G, _K, _D, _B = 4, 8192, 32, 2048
_CC = 0.25
_BB = 128
_NB = _B // _BB


def _vq_kernel(x_ref, w_ref, w2_ref, oh_ref, emb_ref, qst_ref, loss_ref,
               perp_ref, counts_ref):
    b = pl.program_id(1)
    first = jnp.logical_and(pl.program_id(0) == 0, b == 0)

    x = x_ref[0]
    w = w_ref[0]

    x2 = jax.lax.dot_general(x * x, jnp.ones((_D, 1), jnp.float32),
                             (((1,), (0,)), ((), ())),
                             preferred_element_type=jnp.float32)

    x2x = x + x
    _C = 4096
    _NC = _K // _C
    lane = jax.lax.broadcasted_iota(jnp.int32, (1, _C), 1)

    minv = jnp.full((_BB, _C), jnp.inf, jnp.float32)
    mini = jnp.full((_BB, _C), _K, jnp.int32)
    for i in range(_NC):
        wc = w_ref[0, i * _C:(i + 1) * _C, :]
        mm2c = jax.lax.dot_general(x2x, wc, (((1,), (1,)), ((), ())),
                                   preferred_element_type=jnp.float32)
        w2c = w2_ref[0, :, i * _C:(i + 1) * _C]
        dc = (x2 + w2c) - mm2c
        ioc = lane + i * _C
        lt = dc < minv
        minv = jnp.where(lt, dc, minv)
        mini = jnp.where(lt, jnp.broadcast_to(ioc, (_BB, _C)), mini)

    rowmin = jnp.min(minv, axis=1, keepdims=True)
    idx = jnp.min(jnp.where(minv == rowmin, mini, jnp.int32(_K)), axis=1)

    iota = jax.lax.broadcasted_iota(jnp.int32, (1, _K), 1)
    oh = (iota == idx[:, None]).astype(jnp.float32)
    oh_ref[0] = oh

    q = jax.lax.dot_general(oh, w, (((1,), (0,)), ((), ())),
                            preferred_element_type=jnp.float32)
    emb_ref[0] = q
    qst_ref[0] = x + (q - x)

    @pl.when(first)
    def _():
        loss_ref[...] = jnp.zeros((1, 1), jnp.float32)
        perp_ref[...] = jnp.zeros((1, 1), jnp.float32)

    d = q - x
    loss_ref[...] += jnp.sum(d * d).reshape(1, 1)

    colsum = jax.lax.dot_general(jnp.ones((1, _BB), jnp.float32), oh,
                                 (((1,), (0,)), ((), ())),
                                 preferred_element_type=jnp.float32)

    @pl.when(b == 0)
    def _():
        counts_ref[...] = colsum

    @pl.when(b != 0)
    def _():
        counts_ref[...] += colsum

    @pl.when(b == _NB - 1)
    def _():
        p = counts_ref[...] * (1.0 / _B)
        ent = jnp.sum(p * jnp.log(p + 1e-10))
        perp_ref[...] += jnp.exp(-ent).reshape(1, 1)


def kernel(inputs, W):
    xt = jnp.transpose(inputs, (1, 0, 2))
    w2 = jnp.sum(W * W, axis=2)[:, None, :]

    grid = (_G, _NB)
    oh, emb, qst, loss_sum, perp_sum = pl.pallas_call(
        _vq_kernel,
        grid=grid,
        in_specs=[
            pl.BlockSpec((1, _BB, _D), lambda g, b: (g, b, 0)),
            pl.BlockSpec((1, _K, _D), lambda g, b: (g, 0, 0)),
            pl.BlockSpec((1, 1, _K), lambda g, b: (g, 0, 0)),
        ],
        out_specs=[
            pl.BlockSpec((1, _BB, _K), lambda g, b: (g, b, 0)),
            pl.BlockSpec((1, _BB, _D), lambda g, b: (g, b, 0)),
            pl.BlockSpec((1, _BB, _D), lambda g, b: (g, b, 0)),
            pl.BlockSpec((1, 1), lambda g, b: (0, 0)),
            pl.BlockSpec((1, 1), lambda g, b: (0, 0)),
        ],
        out_shape=[
            jax.ShapeDtypeStruct((_G, _B, _K), jnp.float32),
            jax.ShapeDtypeStruct((_G, _B, _D), jnp.float32),
            jax.ShapeDtypeStruct((_G, _B, _D), jnp.float32),
            jax.ShapeDtypeStruct((1, 1), jnp.float32),
            jax.ShapeDtypeStruct((1, 1), jnp.float32),
        ],
        scratch_shapes=[pltpu.VMEM((1, _K), jnp.float32)],
        compiler_params=pltpu.CompilerParams(
            dimension_semantics=("arbitrary", "arbitrary")),
    )(xt, W, w2)

    avg_loss = (loss_sum[0, 0] * ((1.0 + _CC) / (_B * _D))) / _G
    avg_perplexity = perp_sum[0, 0] / _G
    quantized_all = jnp.transpose(qst, (1, 0, 2))
    return (avg_loss, quantized_all, avg_perplexity, emb, oh)
